# Initial kernel scaffold; baseline (speedup 1.0000x reference)
import functools
import jax
import jax.numpy as jnp
from jax import lax
from jax.experimental import pallas as pl
from jax.experimental.pallas import tpu as pltpu
from jax.experimental.pallas import tpu_sc as plsc


def _probe_body(heat_hbm, reg_hbm, out_hbm, slab, hist2, hist1, cval, cntbuf,
                gidx, gout, sh_hist, sem):
    c_idx = lax.axis_index("c")
    s_idx = lax.axis_index("s")
    b = c_idx * 4 + s_idx // 4
    p = s_idx % 4

    lanes = lax.iota(jnp.int32, 16)
    ones = jnp.ones((16,), jnp.int32)

    src0 = b * 192 + p * 48
    pltpu.sync_copy(heat_hbm.at[pl.ds(src0, 48)], slab.at[pl.ds(1, 48)])
    pltpu.sync_copy(heat_hbm.at[pl.ds(jnp.maximum(src0 - 1, 0), 1)],
                    slab.at[pl.ds(0, 1)])

    def row_body(r, acc):
        v = slab[r, pl.ds(16, 16)]
        g = plsc.load_gather(slab, [jnp.full((16,), r, jnp.int32),
                                    jnp.maximum(lanes - 1, 0)])
        m = v >= 0.5
        bucket = (v * 512.0).astype(jnp.int32)
        plsc.addupdate_scatter(hist2, [lanes, bucket], ones)
        excl = plsc.cumsum(m.astype(jnp.int32)) - m.astype(jnp.int32)
        pos = acc + excl
        plsc.store_scatter(cval, [pos], v + g, mask=m)
        pc = plsc.all_reduce_population_count(m)
        return acc + pc

    cnt_vec = lax.fori_loop(0, 49, row_body, jnp.zeros((16,), jnp.int32))

    cntbuf[0] = cnt_vec[0]
    plsc.subcore_barrier()

    def red_body(j, _):
        acc = jnp.zeros((16,), jnp.int32)
        for l in range(16):
            acc = acc + plsc.load_gather(
                hist2, [jnp.full((16,), l, jnp.int32), j * 16 + lanes])
        plsc.store_scatter(hist1, [j * 16 + lanes], acc)
        return 0
    lax.fori_loop(0, 32, red_body, 0)
    pltpu.sync_copy(hist1, sh_hist.at[s_idx])
    plsc.subcore_barrier()

    def suf_body(i, carry):
        jstar, csum = carry
        j0 = (31 - i) * 16
        cnt16 = plsc.load_gather(hist1, [j0 + lanes])
        s = lax.rev(plsc.cumsum(lax.rev(cnt16, (0,))), (0,)) + csum
        m = s >= 100
        pc = plsc.all_reduce_population_count(m)
        jc = jnp.where(pc > 0, j0 + pc - 1, -1)
        csum = csum + jnp.sum(cnt16)
        return jnp.maximum(jstar, jc), csum
    jstar, _ = lax.fori_loop(0, 32, suf_body,
                             (jnp.full((16,), -1, jnp.int32),
                              jnp.zeros((), jnp.int32)))

    @pl.when(p == 0)
    def _():
        base = b * 8 * 122880
        for ch in range(2):
            addr = base + ch * 122880 + lanes * 640
            plsc.store_scatter(gidx, [ch * 128 + lanes], addr)
        for ch in range(2):
            cp = pltpu.async_copy(reg_hbm.at[gidx.at[ch]], gout.at[ch], sem)
            cp.wait()
        fo = gout[0, pl.ds(0, 16), 0] + jstar.astype(jnp.float32)
        out_hbm_row = out_hbm.at[b]
        pltpu.sync_copy(cval.at[pl.ds(0, 16)], out_hbm_row.at[pl.ds(0, 16)])
        cval[pl.ds(16, 16)] = fo
        pltpu.sync_copy(cval.at[pl.ds(16, 16)], out_hbm_row.at[pl.ds(16, 16)])


def kernel(pred_heatmap, pred_regression, trans_mat, K_mat, size):
    heat2d = pred_heatmap.reshape(8 * 3 * 192, 640)
    regflat = pred_regression.reshape(-1, 1)

    mesh = plsc.VectorSubcoreMesh(core_axis_name="c", subcore_axis_name="s")
    run = pl.kernel(
        _probe_body,
        out_type=jax.ShapeDtypeStruct((8, 128), jnp.float32),
        mesh=mesh,
        scratch_types=[
            pltpu.VMEM((50, 640), jnp.float32),
            pltpu.VMEM((16, 512), jnp.int32),
            pltpu.VMEM((512,), jnp.int32),
            pltpu.VMEM((512,), jnp.float32),
            pltpu.VMEM((16,), jnp.int32),
            pltpu.VMEM((8, 128), jnp.int32),
            pltpu.VMEM((8, 128, 1), jnp.float32),
            pltpu.VMEM_SHARED((16, 512), jnp.int32),
            pltpu.SemaphoreType.DMA,
        ],
    )
    out = run(heat2d, regflat)
    dummy = out[:, :100]
    z = jnp.zeros((8, 100, 1), jnp.float32) + dummy[..., None]
    return (jnp.zeros((8, 100, 7), jnp.float32) + dummy[..., None],
            z, z, jnp.zeros((8, 100, 4), jnp.float32), z)


# speedup vs baseline: 5.7688x; 5.7688x over previous
import functools

import jax
import jax.numpy as jnp
from jax import lax
from jax.experimental import pallas as pl
from jax.experimental.pallas import tpu as pltpu
from jax.experimental.pallas import tpu_sc as plsc

B = 8
C = 3
H = 192
W = 640
HW = H * W
CHW = C * HW
NB = 1024
CAP = 512
MCAP = 4 * CAP
KDET = 100
NVREG = W // 16
ROWS = H // 4
BIGI = 1 << 30

PI = 3.14159265358979323846

_ATAN_C = (0.9999999828647295, -0.3333319654947795, 0.19996761628871623,
           -0.14250134536882025, 0.10891953602934044, -0.08252553527065536,
           0.05567457385216619, -0.029126338687892307, 0.009906944500958914,
           -0.00158530861159642)

_SX = (-0.5, 0.5, 0.5, 0.5, 0.5, -0.5, -0.5, -0.5)
_SY = (-1.0, -1.0, 0.0, 0.0, -1.0, -1.0, 0.0, 0.0)
_SZ = (-0.5, -0.5, -0.5, 0.5, 0.5, 0.5, 0.5, -0.5)

_DIM_REF = ((3.88, 1.63, 1.53), (1.78, 1.70, 0.58), (0.88, 0.76, 1.76))


def _sc_body(heat_hbm, reg_hbm, svals_hbm, sidx_hbm, pois_hbm,
             slab, hbuf, hist2, hist1, hist4, cval, cidx, c16,
             mrgv, mrgi, svbuf, sibuf, tmpv, tmpi, gidx, gout,
             sh_hist, sh_cval, sh_cidx, sh_cnt, sem):
    c_core = lax.axis_index("c")
    s_idx = lax.axis_index("s")
    b = c_core * 4 + s_idx // 4
    p = s_idx % 4
    s0 = (s_idx // 4) * 4

    lanes = lax.iota(jnp.int32, 16)
    ones_i = jnp.ones((16,), jnp.int32)

    def dma_slab(c):
        base = (b * 3 + c) * H
        r0 = base + p * ROWS
        cp0 = pltpu.async_copy(
            heat_hbm.at[pl.ds(pl.multiple_of(r0 * W, W), ROWS * W)],
            slab.at[pl.ds(W, ROWS * W)], sem)
        rt = base + jnp.maximum(p * ROWS - 1, 0)
        cp1 = pltpu.async_copy(
            heat_hbm.at[pl.ds(pl.multiple_of(rt * W, W), W)],
            slab.at[pl.ds(0, W)], sem)
        rb = base + jnp.minimum(p * ROWS + ROWS, H - 1)
        cp2 = pltpu.async_copy(
            heat_hbm.at[pl.ds(pl.multiple_of(rb * W, W), W)],
            slab.at[pl.ds((ROWS + 1) * W, W)], sem)
        cp0.wait()
        cp1.wait()
        cp2.wait()

    def compute_h():
        def hrow(r, _):
            off = r * W
            for k in range(NVREG):
                x0 = k * 16
                vc = slab[pl.ds(off + x0, 16)]
                if k == 0:
                    vl = plsc.load_gather(
                        slab, [off + jnp.maximum(lanes - 1, 0)])
                else:
                    vl = slab[pl.ds(off + x0 - 1, 16)]
                if k == NVREG - 1:
                    vr = plsc.load_gather(
                        slab, [off + jnp.minimum(x0 + lanes + 1, W - 1)])
                else:
                    vr = slab[pl.ds(off + x0 + 1, 16)]
                hbuf[pl.ds(off + x0, 16)] = jnp.maximum(jnp.maximum(vl, vc),
                                                        vr)
            return 0
        lax.fori_loop(0, ROWS + 2, hrow, 0)

    def zero_hist(i, _):
        hist2[pl.ds(i * 16, 16)] = jnp.zeros((16,), jnp.int32)
        return 0
    lax.fori_loop(0, 16 * NB // 16, zero_hist, 0)

    lane_hist_base = lanes * NB

    for c in range(C):
        dma_slab(c)
        compute_h()

        def hist_row(yl, _):
            off = yl * W
            for k in range(NVREG):
                x0 = k * 16
                v = slab[pl.ds(off + x0, 16)]
                hm = jnp.maximum(
                    jnp.maximum(hbuf[pl.ds(off - W + x0, 16)],
                                hbuf[pl.ds(off + x0, 16)]),
                    hbuf[pl.ds(off + W + x0, 16)])
                nms = jnp.where(v == hm, v, 0.0)
                bkt = jnp.clip((nms * float(NB)).astype(jnp.int32), 0, NB - 1)
                plsc.addupdate_scatter(hist2, [lane_hist_base + bkt], ones_i)
            return 0
        lax.fori_loop(1, ROWS + 1, hist_row, 0)

    def hist_red(j, _):
        acc = jnp.zeros((16,), jnp.int32)
        for l in range(16):
            acc = acc + hist2[pl.ds(l * NB + j * 16, 16)]
        hist1[pl.ds(j * 16, 16)] = acc
        return 0
    lax.fori_loop(0, NB // 16, hist_red, 0)

    pltpu.sync_copy(hist1,
                    sh_hist.at[pl.ds(pl.multiple_of(s_idx * NB, NB), NB)])
    plsc.subcore_barrier()

    pltpu.sync_copy(sh_hist.at[pl.ds(pl.multiple_of(s0 * NB, NB), 4 * NB)],
                    hist4)

    def suf_body(i, carry):
        jstar, csum = carry
        j0 = (NB // 16 - 1 - i) * 16
        cnt16 = (hist4[pl.ds(j0, 16)] + hist4[pl.ds(NB + j0, 16)]
                 + hist4[pl.ds(2 * NB + j0, 16)]
                 + hist4[pl.ds(3 * NB + j0, 16)])
        suf = lax.rev(plsc.cumsum(lax.rev(cnt16, (0,))), (0,)) + csum
        m = suf >= KDET
        pc = plsc.all_reduce_population_count(m)
        jc = jnp.where(pc > 0, j0 + pc - 1, -1)
        return jnp.maximum(jstar, jc), csum + jnp.sum(cnt16)

    jstar, _ = lax.fori_loop(
        0, NB // 16, suf_body,
        (jnp.full((16,), -1, jnp.int32), jnp.zeros((), jnp.int32)))
    thrf = jstar.astype(jnp.float32)

    def prefill(i, _):
        cval[pl.ds(i * 16, 16)] = jnp.full((16,), -1.0, jnp.float32)
        cidx[pl.ds(i * 16, 16)] = jnp.full((16,), BIGI, jnp.int32)
        return 0
    lax.fori_loop(0, CAP // 16, prefill, 0)

    cnt = jnp.zeros((), jnp.int32)
    for c in range(C):
        dma_slab(c)
        compute_h()

        def coll_row(yl, cnt):
            off = yl * W
            fbase = c * HW + (p * ROWS + yl - 1) * W
            for k in range(NVREG):
                x0 = k * 16
                v = slab[pl.ds(off + x0, 16)]
                hm = jnp.maximum(
                    jnp.maximum(hbuf[pl.ds(off - W + x0, 16)],
                                hbuf[pl.ds(off + x0, 16)]),
                    hbuf[pl.ds(off + W + x0, 16)])
                nms = jnp.where(v == hm, v, 0.0)
                m = nms * float(NB) >= thrf
                pc = plsc.all_reduce_population_count(m)[0]
                cnte = jnp.minimum(cnt, CAP - 16)
                plsc.store_compressed(cval.at[pl.ds(cnte, 16)], nms, mask=m)
                plsc.store_compressed(cidx.at[pl.ds(cnte, 16)],
                                      fbase + x0 + lanes, mask=m)
                cnt = cnt + pc
            return cnt
        cnt = lax.fori_loop(1, ROWS + 1, coll_row, cnt)

    cnt = jnp.minimum(cnt, CAP)
    c16[...] = jnp.zeros((16,), jnp.int32) + cnt
    pltpu.sync_copy(cval,
                    sh_cval.at[pl.ds(pl.multiple_of(s_idx * CAP, CAP), CAP)])
    pltpu.sync_copy(cidx,
                    sh_cidx.at[pl.ds(pl.multiple_of(s_idx * CAP, CAP), CAP)])
    pltpu.sync_copy(c16, sh_cnt.at[pl.ds(pl.multiple_of(s_idx * 16, 16), 16)])
    plsc.subcore_barrier()

    @pl.when(p == 0)
    def _():
        def mprefill(i, _):
            mrgv[pl.ds(i * 16, 16)] = jnp.full((16,), -1.0, jnp.float32)
            mrgi[pl.ds(i * 16, 16)] = jnp.full((16,), BIGI, jnp.int32)
            return 0
        lax.fori_loop(0, MCAP // 16, mprefill, 0)

        tot = jnp.zeros((), jnp.int32)
        for j in range(4):
            pltpu.sync_copy(
                sh_cval.at[pl.ds(pl.multiple_of((s0 + j) * CAP, CAP), CAP)],
                tmpv)
            pltpu.sync_copy(
                sh_cidx.at[pl.ds(pl.multiple_of((s0 + j) * CAP, CAP), CAP)],
                tmpi)
            pltpu.sync_copy(
                sh_cnt.at[pl.ds(pl.multiple_of((s0 + j) * 16, 16), 16)], c16)
            cj = c16[...][0]
            nj = (cj + 15) // 16

            def cpv(i, tot):
                mrgv[pl.ds(tot + i * 16, 16)] = tmpv[pl.ds(i * 16, 16)]
                mrgi[pl.ds(tot + i * 16, 16)] = tmpi[pl.ds(i * 16, 16)]
                return tot
            lax.fori_loop(0, nj, cpv, tot)
            tot = tot + nj * 16

        nv = tot // 16

        def selfill(i, _):
            svbuf[pl.ds(i * 16, 16)] = jnp.zeros((16,), jnp.float32)
            sibuf[pl.ds(i * 16, 16)] = lanes + i * 16 * W
            return 0
        lax.fori_loop(0, 8, selfill, 0)

        def sel(kk, prev):
            def scan(i, car):
                mv, fv = car
                off = i * 16
                v = mrgv[pl.ds(off, 16)]
                fi = mrgi[pl.ds(off, 16)]
                v = jnp.where(fi == prev, -2.0, v)
                mrgv[pl.ds(off, 16)] = v
                better = (v > mv) | ((v == mv) & (fi < fv))
                return (jnp.where(better, v, mv), jnp.where(better, fi, fv))
            mv, fv = lax.fori_loop(
                0, nv, scan,
                (jnp.full((16,), -3.0, jnp.float32),
                 jnp.full((16,), BIGI, jnp.int32)))
            mx = jnp.max(mv)
            istar = jnp.min(jnp.where(mv == mx, fv, BIGI))
            kv = jnp.full((16,), kk, jnp.int32)
            plsc.store_scatter(svbuf, [kv],
                               jnp.zeros((16,), jnp.float32) + mx,
                               mask=lanes == 0)
            plsc.store_scatter(sibuf, [kv],
                               jnp.zeros((16,), jnp.int32) + istar,
                               mask=lanes == 0)
            return istar
        lax.fori_loop(0, KDET, sel, jnp.int32(-7))

        for t in range(8):
            fi = sibuf[pl.ds(t * 16, 16)]
            cls = fi // HW
            sp = fi - cls * HW
            for ch in range(8):
                plsc.store_scatter(gidx, [ch * 128 + t * 16 + lanes],
                                   (b * 8 + ch) * HW + sp)
        cps = []
        for ch in range(8):
            cps.append(pltpu.async_copy(
                reg_hbm.at[gidx.at[pl.ds(ch * 128, 128)]],
                gout.at[pl.ds(ch * 128, 128)], sem))
        for cp in cps:
            cp.wait()

        pltpu.sync_copy(
            svbuf, svals_hbm.at[pl.ds(pl.multiple_of(b * 128, 128), 128)])
        pltpu.sync_copy(
            sibuf, sidx_hbm.at[pl.ds(pl.multiple_of(b * 128, 128), 128)])
        pltpu.sync_copy(
            gout, pois_hbm.at[pl.ds(pl.multiple_of(b * 1024, 1024), 1024)])


def _bf(x):
    return x.astype(jnp.bfloat16).astype(jnp.float32)


def _atan(x):
    s = jnp.sign(x)
    a = jnp.abs(x)
    inv = a > 1.0
    z = jnp.where(inv, 1.0 / jnp.maximum(a, 1e-30), a)
    u = z * z
    pacc = jnp.full_like(z, _ATAN_C[-1])
    for cc in _ATAN_C[-2::-1]:
        pacc = pacc * u + cc
    pv = z * pacc
    return jnp.where(inv, PI / 2 - pv, pv) * s


def _tc_body(sv_ref, si_ref, po_ref, par_ref, out_ref):
    fi = si_ref[...]
    cls = fi // HW
    sp = fi - cls * HW
    ysi = sp // W
    xs = (sp - ysi * W).astype(jnp.float32)
    ys = ysi.astype(jnp.float32)

    def par(i):
        return par_ref[0, i]

    k00, k01, k02 = par(0), par(1), par(2)
    k10, k11, k12 = par(3), par(4), par(5)
    k20, k21, k22 = par(6), par(7), par(8)
    t00, t01, t02 = par(9), par(10), par(11)
    t10, t11, t12 = par(12), par(13), par(14)
    t20, t21, t22 = par(15), par(16), par(17)
    sz0, sz1 = par(18), par(19)

    def inv3(a00, a01, a02, a10, a11, a12, a20, a21, a22):
        det = (a00 * (a11 * a22 - a12 * a21)
               - a01 * (a10 * a22 - a12 * a20)
               + a02 * (a10 * a21 - a11 * a20))
        r = 1.0 / det
        return ((a11 * a22 - a12 * a21) * r, (a02 * a21 - a01 * a22) * r,
                (a01 * a12 - a02 * a11) * r, (a12 * a20 - a10 * a22) * r,
                (a00 * a22 - a02 * a20) * r, (a02 * a10 - a00 * a12) * r,
                (a10 * a21 - a11 * a20) * r, (a01 * a20 - a00 * a21) * r,
                (a00 * a11 - a01 * a10) * r)

    ti00, ti01, ti02, ti10, ti11, ti12, ti20, ti21, ti22 = inv3(
        t00, t01, t02, t10, t11, t12, t20, t21, t22)
    ki00, ki01, ki02, ki10, ki11, ki12, ki20, ki21, ki22 = inv3(
        k00, k01, k02, k10, k11, k12, k20, k21, k22)

    dep_off = po_ref[:, 0, :]
    off_x = po_ref[:, 1, :]
    off_y = po_ref[:, 2, :]
    dim0 = po_ref[:, 3, :]
    dim1 = po_ref[:, 4, :]
    dim2 = po_ref[:, 5, :]
    ori0 = po_ref[:, 6, :]
    ori1 = po_ref[:, 7, :]

    dep = dep_off * 16.32 + 28.01
    px = _bf(xs + off_x)
    py = _bf(ys + off_y)
    bt00, bt01, bt02 = _bf(ti00), _bf(ti01), _bf(ti02)
    bt10, bt11, bt12 = _bf(ti10), _bf(ti11), _bf(ti12)
    bt20, bt21, bt22 = _bf(ti20), _bf(ti21), _bf(ti22)
    ax = (bt00 * px + bt01 * py + bt02) * dep
    ay = (bt10 * px + bt11 * py + bt12) * dep
    az = (bt20 * px + bt21 * py + bt22) * dep
    ax, ay, az = _bf(ax), _bf(ay), _bf(az)
    bk00, bk01, bk02 = _bf(ki00), _bf(ki01), _bf(ki02)
    bk10, bk11, bk12 = _bf(ki10), _bf(ki11), _bf(ki12)
    bk20, bk21, bk22 = _bf(ki20), _bf(ki21), _bf(ki22)
    lx = bk00 * ax + bk01 * ay + bk02 * az
    ly = bk10 * ax + bk11 * ay + bk12 * az
    lz = bk20 * ax + bk21 * ay + bk22 * az

    clsf = cls.astype(jnp.float32)

    def dref(j):
        return jnp.where(cls == 0, _DIM_REF[0][j],
                         jnp.where(cls == 1, _DIM_REF[1][j], _DIM_REF[2][j]))

    d0 = jnp.exp(dim0) * dref(0)
    d1 = jnp.exp(dim1) * dref(1)
    d2 = jnp.exp(dim2) * dref(2)
    ly = ly + d1 * 0.5

    rays = _atan(lx / (lz + 1e-7))
    alphas = _atan(ori0 / (ori1 + 1e-7))
    alphas = jnp.where(ori1 >= 0, alphas - PI / 2, alphas + PI / 2)
    rotys = alphas + rays
    rotys = jnp.where(rotys > PI, rotys - 2 * PI, rotys)
    rotys = jnp.where(rotys < -PI, rotys + 2 * PI, rotys)

    cosr = jnp.cos(rotys)
    sinr = jnp.sin(rotys)
    bc, bs = _bf(cosr), _bf(sinr)
    bK00, bK01, bK02 = _bf(k00), _bf(k01), _bf(k02)
    bK10, bK11, bK12 = _bf(k10), _bf(k11), _bf(k12)
    bK20, bK21, bK22 = _bf(k20), _bf(k21), _bf(k22)

    big = jnp.float32(1e30)
    xmn, xmx, ymn, ymx = big, -big, big, -big
    for k in range(8):
        xk = _bf(_SX[k] * d0)
        yk = _bf(_SY[k] * d1)
        zk = _bf(_SZ[k] * d2)
        cx = _bf(bc * xk + bs * zk + lx)
        cy = _bf(yk + ly)
        cz = _bf(-bs * xk + bc * zk + lz)
        iu = bK00 * cx + bK01 * cy + bK02 * cz
        iv = bK10 * cx + bK11 * cy + bK12 * cz
        iw = bK20 * cx + bK21 * cy + bK22 * cz
        u = iu / iw
        v = iv / iw
        xmn = jnp.minimum(xmn, u)
        xmx = jnp.maximum(xmx, u)
        ymn = jnp.minimum(ymn, v)
        ymx = jnp.maximum(ymx, v)

    xmn = jnp.clip(xmn, 0.0, sz0)
    xmx = jnp.clip(xmx, 0.0, sz0)
    ymn = jnp.clip(ymn, 0.0, sz1)
    ymx = jnp.clip(ymx, 0.0, sz1)

    zeros = jnp.zeros_like(lx)
    out_ref[:, 0, :] = lx
    out_ref[:, 1, :] = ly
    out_ref[:, 2, :] = lz
    out_ref[:, 3, :] = d1
    out_ref[:, 4, :] = d2
    out_ref[:, 5, :] = d0
    out_ref[:, 6, :] = rotys
    out_ref[:, 7, :] = clsf
    out_ref[:, 8, :] = alphas
    out_ref[:, 9, :] = xmn
    out_ref[:, 10, :] = ymn
    out_ref[:, 11, :] = xmx
    out_ref[:, 12, :] = ymx
    out_ref[:, 13, :] = sv_ref[...]
    out_ref[:, 14, :] = zeros
    out_ref[:, 15, :] = zeros


def kernel(pred_heatmap, pred_regression, trans_mat, K_mat, size):
    heat1d = pred_heatmap.reshape(-1)
    reg1d = pred_regression.reshape(-1)

    mesh = plsc.VectorSubcoreMesh(core_axis_name="c", subcore_axis_name="s")
    sc = pl.kernel(
        _sc_body,
        out_type=(jax.ShapeDtypeStruct((B * 128,), jnp.float32),
                  jax.ShapeDtypeStruct((B * 128,), jnp.int32),
                  jax.ShapeDtypeStruct((B * 1024,), jnp.float32)),
        mesh=mesh,
        compiler_params=pltpu.CompilerParams(needs_layout_passes=False),
        scratch_types=[
            pltpu.VMEM(((ROWS + 2) * W,), jnp.float32),
            pltpu.VMEM(((ROWS + 2) * W,), jnp.float32),
            pltpu.VMEM((16 * NB,), jnp.int32),
            pltpu.VMEM((NB,), jnp.int32),
            pltpu.VMEM((4 * NB,), jnp.int32),
            pltpu.VMEM((CAP,), jnp.float32),
            pltpu.VMEM((CAP,), jnp.int32),
            pltpu.VMEM((16,), jnp.int32),
            pltpu.VMEM((MCAP,), jnp.float32),
            pltpu.VMEM((MCAP,), jnp.int32),
            pltpu.VMEM((128,), jnp.float32),
            pltpu.VMEM((128,), jnp.int32),
            pltpu.VMEM((CAP,), jnp.float32),
            pltpu.VMEM((CAP,), jnp.int32),
            pltpu.VMEM((1024,), jnp.int32),
            pltpu.VMEM((1024,), jnp.float32),
            pltpu.VMEM_SHARED((16 * NB,), jnp.int32),
            pltpu.VMEM_SHARED((16 * CAP,), jnp.float32),
            pltpu.VMEM_SHARED((16 * CAP,), jnp.int32),
            pltpu.VMEM_SHARED((16 * 16,), jnp.int32),
            pltpu.SemaphoreType.DMA,
        ],
    )
    svals, sidx, pois = sc(heat1d, reg1d)
    svals2 = svals.reshape(B, 128)
    sidx2 = sidx.reshape(B, 128)
    pois3 = pois.reshape(B, 8, 128)

    par = jnp.concatenate([
        K_mat.reshape(-1), trans_mat.reshape(-1), size.reshape(-1),
        jnp.zeros((12,), jnp.float32)
    ]).reshape(1, 32)

    out = pl.pallas_call(
        _tc_body,
        out_shape=jax.ShapeDtypeStruct((B, 16, 128), jnp.float32),
        in_specs=[
            pl.BlockSpec(memory_space=pltpu.MemorySpace.VMEM),
            pl.BlockSpec(memory_space=pltpu.MemorySpace.VMEM),
            pl.BlockSpec(memory_space=pltpu.MemorySpace.VMEM),
            pl.BlockSpec(memory_space=pltpu.MemorySpace.SMEM),
        ],
        out_specs=pl.BlockSpec(memory_space=pltpu.MemorySpace.VMEM),
    )(svals2, sidx2, pois3, par)

    locs = jnp.transpose(out[:, 0:3, :KDET], (0, 2, 1))
    dims_r = jnp.transpose(out[:, 3:6, :KDET], (0, 2, 1))
    rotys = jnp.transpose(out[:, 6:7, :KDET], (0, 2, 1))
    pred_boxes = jnp.concatenate([locs, dims_r, rotys], axis=2)
    scores_o = out[:, 13, :KDET][..., None]
    clses_o = out[:, 7, :KDET][..., None]
    box2d_o = jnp.stack([out[:, 9, :KDET], out[:, 10, :KDET],
                         out[:, 11, :KDET], out[:, 12, :KDET]], axis=2)
    alphas_o = out[:, 8, :KDET][..., None]
    return pred_boxes, scores_o, clses_o, box2d_o, alphas_o
